# baseline (speedup 1.0000x reference)
import functools

import jax
import jax.numpy as jnp
from jax import lax
from jax.experimental import pallas as pl
from jax.experimental.pallas import tpu as pltpu
from jax.experimental.pallas import tpu_sc as plsc

N_NODES = 10000
N_EDGES = 320000
D = 128
DH = D // 2

NC = 2
NS = 16
NW = NC * NS
EPW = N_EDGES // NW
K = 80
NCH = EPW // K
ZCH = 200
NZC = N_NODES // ZCH


def _make_sc_segsum(with_cnt: bool):
    mesh = plsc.VectorSubcoreMesh(core_axis_name="c", subcore_axis_name="s")

    out_type = [jax.ShapeDtypeStruct((NC, N_NODES, DH), jnp.float32),
                jax.ShapeDtypeStruct((NC, N_NODES, DH), jnp.float32)]
    scratch = [
        pltpu.VMEM((NCH, K), jnp.int32),
        pltpu.VMEM((NCH, K), jnp.int32),
        pltpu.VMEM((4, K, DH), jnp.float32),
        pltpu.VMEM((ZCH, DH), jnp.float32),
        pltpu.VMEM_SHARED((N_NODES, DH), jnp.float32),
        pltpu.SemaphoreType.DMA((4,)),
        pltpu.SemaphoreType.DMA((4,)),
    ]
    if with_cnt:
        out_type.append(jax.ShapeDtypeStruct((NC, N_NODES), jnp.float32))
        scratch += [
            pltpu.VMEM((K,), jnp.float32),
            pltpu.VMEM((N_NODES,), jnp.float32),
            pltpu.VMEM_SHARED((N_NODES,), jnp.float32),
        ]

    @functools.partial(
        pl.kernel, mesh=mesh, out_type=out_type, scratch_types=scratch,
        compiler_params=pltpu.CompilerParams(use_tc_tiling_on_sc=False))
    def body(feat_lo, feat_hi, src3, dst3, *rest):
        if with_cnt:
            (out_lo, out_hi, cntp, srcv, dstv, rows, zbuf, acc, gsem, ssem,
             onesv, cbuf, cacc) = rest
        else:
            (out_lo, out_hi, srcv, dstv, rows, zbuf, acc, gsem, ssem) = rest
        c = lax.axis_index("c")
        s = lax.axis_index("s")
        wid = s * NC + c

        pltpu.sync_copy(src3.at[wid], srcv)
        pltpu.sync_copy(dst3.at[wid], dstv)

        zv = jnp.zeros((16,), jnp.float32)

        def zero_zbuf(i, _):
            zbuf[i // (DH // 16), pl.ds((i % (DH // 16)) * 16, 16)] = zv
            return 0
        lax.fori_loop(0, ZCH * (DH // 16), zero_zbuf, 0)

        if with_cnt:
            def fill_ones(i, _):
                onesv[pl.ds(i * 16, 16)] = jnp.ones((16,), jnp.float32)
                return 0
            lax.fori_loop(0, K // 16, fill_ones, 0)

            @pl.when(s == 0)
            def _():
                def zero_cbuf(i, _):
                    cbuf[pl.ds(i * 16, 16)] = zv
                    return 0
                lax.fori_loop(0, N_NODES // 16, zero_cbuf, 0)
                pltpu.sync_copy(cbuf, cacc)

        for p, (feat, outp) in enumerate(((feat_lo, out_lo),
                                          (feat_hi, out_hi))):
            def zero_acc(j, _):
                ch = s + j * NS

                @pl.when(ch < NZC)
                def _():
                    pltpu.sync_copy(zbuf, acc.at[pl.ds(ch * ZCH, ZCH)])
                return 0
            lax.fori_loop(0, (NZC + NS - 1) // NS, zero_acc, 0)

            plsc.subcore_barrier()

            first = with_cnt and p == 0

            pltpu.async_copy(feat.at[srcv.at[0]], rows.at[0], gsem.at[0])
            pltpu.async_copy(feat.at[srcv.at[1]], rows.at[1], gsem.at[1])

            def chunk(i, _):
                b = i % 4
                rb = rows.at[b]
                pltpu.make_async_copy(feat.at[srcv.at[i]], rb,
                                      gsem.at[b]).wait()
                pltpu.async_copy(rb, acc.at[dstv.at[i]], ssem.at[b],
                                 add=True)
                if first:
                    pltpu.sync_copy(onesv, cacc.at[dstv.at[i]], add=True)

                @pl.when(i < NCH - 2)
                def _():
                    b2 = (i + 2) % 4
                    rb2 = rows.at[b2]

                    @pl.when(i >= 2)
                    def _():
                        pltpu.make_async_copy(
                            rb2, acc.at[dstv.at[i]], ssem.at[b2]).wait()
                    pltpu.async_copy(feat.at[srcv.at[i + 2]], rb2,
                                     gsem.at[b2])
                return 0
            lax.fori_loop(0, NCH, chunk, 0)

            for t in range(NCH - 4, NCH):
                pltpu.make_async_copy(rows.at[t % 4], acc.at[dstv.at[0]],
                                      ssem.at[t % 4]).wait()

            plsc.subcore_barrier()

            def export(j, _):
                ch = s + j * NS

                @pl.when(ch < NZC)
                def _():
                    r0 = ch * ZCH
                    pltpu.sync_copy(acc.at[pl.ds(r0, ZCH)], zbuf)
                    pltpu.sync_copy(zbuf, outp.at[c, pl.ds(r0, ZCH)])
                return 0
            lax.fori_loop(0, (NZC + NS - 1) // NS, export, 0)

            if p == 0:
                lax.fori_loop(0, ZCH * (DH // 16), zero_zbuf, 0)

        if with_cnt:
            @pl.when(s == 0)
            def _():
                pltpu.sync_copy(cacc, cbuf)
                pltpu.sync_copy(cbuf, cntp.at[c])

    return body


_sc_segsum_cnt = _make_sc_segsum(True)
_sc_segsum = _make_sc_segsum(False)

BR = 1000

_row_spec = pl.BlockSpec((BR, D), lambda i: (i, 0))
_half_spec = pl.BlockSpec((BR, DH), lambda i: (i, 0))
_cnt_spec = pl.BlockSpec((BR, 1), lambda i: (i, 0))
_w_spec = pl.BlockSpec((D, D), lambda i: (0, 0))
_b_spec = pl.BlockSpec((1, D), lambda i: (0, 0))


def _mean_from(lo0_ref, lo1_ref, hi0_ref, hi1_ref, c0_ref, c1_ref):
    cnt = jnp.maximum(c0_ref[...] + c1_ref[...], 1.0)
    agg = jnp.concatenate([lo0_ref[...] + lo1_ref[...],
                           hi0_ref[...] + hi1_ref[...]], axis=1)
    return agg / cnt


def _mid_body(lo0_ref, lo1_ref, hi0_ref, hi1_ref, x_ref, c0_ref, c1_ref,
              wl_ref, wr1_ref, b1_ref, wr2_ref, b2_ref,
              hlo_ref, hhi_ref, t2_ref):
    mean = _mean_from(lo0_ref, lo1_ref, hi0_ref, hi1_ref, c0_ref, c1_ref)
    t1 = jnp.dot(x_ref[...], wr1_ref[...],
                 preferred_element_type=jnp.float32) + b1_ref[...]
    h = jnp.maximum(jnp.dot(mean, wl_ref[...],
                            preferred_element_type=jnp.float32) + t1, 0.0)
    hlo_ref[...] = h[:, :DH]
    hhi_ref[...] = h[:, DH:]
    t2_ref[...] = jnp.dot(h, wr2_ref[...],
                          preferred_element_type=jnp.float32) + b2_ref[...]


def _mid(lo0, lo1, hi0, hi1, x, c0, c1, W1_l, W1_r, b1, W2_r, b2):
    return pl.pallas_call(
        _mid_body,
        grid=(N_NODES // BR,),
        in_specs=[_half_spec, _half_spec, _half_spec, _half_spec, _row_spec,
                  _cnt_spec, _cnt_spec, _w_spec, _w_spec, _b_spec,
                  _w_spec, _b_spec],
        out_specs=[_half_spec, _half_spec, _row_spec],
        out_shape=[jax.ShapeDtypeStruct((N_NODES, DH), jnp.float32),
                   jax.ShapeDtypeStruct((N_NODES, DH), jnp.float32),
                   jax.ShapeDtypeStruct((N_NODES, D), jnp.float32)],
    )(lo0, lo1, hi0, hi1, x, c0, c1, W1_l, W1_r, b1.reshape(1, D),
      W2_r, b2.reshape(1, D))


def _final_body(lo0_ref, lo1_ref, hi0_ref, hi1_ref, t_ref, c0_ref, c1_ref,
                wl_ref, o_ref):
    mean = _mean_from(lo0_ref, lo1_ref, hi0_ref, hi1_ref, c0_ref, c1_ref)
    o_ref[...] = jnp.dot(mean, wl_ref[...],
                         preferred_element_type=jnp.float32) + t_ref[...]


def _final(lo0, lo1, hi0, hi1, t, c0, c1, W_l):
    return pl.pallas_call(
        _final_body,
        grid=(N_NODES // BR,),
        in_specs=[_half_spec, _half_spec, _half_spec, _half_spec, _row_spec,
                  _cnt_spec, _cnt_spec, _w_spec],
        out_specs=_row_spec,
        out_shape=jax.ShapeDtypeStruct((N_NODES, D), jnp.float32),
    )(lo0, lo1, hi0, hi1, t, c0, c1, W_l)


def kernel(x, edge_index, W1_l, b1, W1_r, W2_l, b2, W2_r):
    src = edge_index[0].astype(jnp.int32).reshape(NW, NCH, K)
    dst = edge_index[1].astype(jnp.int32).reshape(NW, NCH, K)
    x_lo = x[:, :DH]
    x_hi = x[:, DH:]

    a_lo, a_hi, cnt = _sc_segsum_cnt(x_lo, x_hi, src, dst)
    c0 = cnt[0].reshape(N_NODES, 1)
    c1 = cnt[1].reshape(N_NODES, 1)
    h_lo, h_hi, t2 = _mid(a_lo[0], a_lo[1], a_hi[0], a_hi[1], x, c0, c1,
                          W1_l, W1_r, b1, W2_r, b2)
    b_lo, b_hi = _sc_segsum(h_lo, h_hi, src, dst)
    out = _final(b_lo[0], b_lo[1], b_hi[0], b_hi[1], t2, c0, c1, W2_l)
    return out


# speedup vs baseline: 1.1658x; 1.1658x over previous
import functools

import jax
import jax.numpy as jnp
from jax import lax
from jax.experimental import pallas as pl
from jax.experimental.pallas import tpu as pltpu
from jax.experimental.pallas import tpu_sc as plsc

N_NODES = 10000
N_EDGES = 320000
D = 128
DH = D // 2

NC = 2
NS = 16
NW = NC * NS
EPW = N_EDGES // NW
K = 80
NCH = EPW // K
ZCH = 200
NZC = N_NODES // ZCH


def _make_sc_segsum(with_cnt: bool):
    mesh = plsc.VectorSubcoreMesh(core_axis_name="c", subcore_axis_name="s")

    out_type = [jax.ShapeDtypeStruct((NC, N_NODES, DH), jnp.float32),
                jax.ShapeDtypeStruct((NC, N_NODES, DH), jnp.float32)]
    scratch = [
        pltpu.VMEM((NCH, K), jnp.int32),
        pltpu.VMEM((NCH, K), jnp.int32),
        pltpu.VMEM((8, K, DH), jnp.float32),
        pltpu.VMEM((ZCH, DH), jnp.float32),
        pltpu.VMEM_SHARED((N_NODES, DH), jnp.float32),
        pltpu.SemaphoreType.DMA((8,)),
        pltpu.SemaphoreType.DMA((8,)),
    ]
    if with_cnt:
        out_type.append(jax.ShapeDtypeStruct((NC, N_NODES), jnp.float32))
        scratch += [
            pltpu.VMEM((K,), jnp.float32),
            pltpu.VMEM((N_NODES,), jnp.float32),
            pltpu.VMEM_SHARED((N_NODES,), jnp.float32),
        ]

    @functools.partial(
        pl.kernel, mesh=mesh, out_type=out_type, scratch_types=scratch,
        compiler_params=pltpu.CompilerParams(use_tc_tiling_on_sc=False))
    def body(feat_lo, feat_hi, src3, dst3, *rest):
        if with_cnt:
            (out_lo, out_hi, cntp, srcv, dstv, rows, zbuf, acc, gsem, ssem,
             onesv, cbuf, cacc) = rest
        else:
            (out_lo, out_hi, srcv, dstv, rows, zbuf, acc, gsem, ssem) = rest
        c = lax.axis_index("c")
        s = lax.axis_index("s")
        wid = s * NC + c

        pltpu.sync_copy(src3.at[wid], srcv)
        pltpu.sync_copy(dst3.at[wid], dstv)

        zv = jnp.zeros((16,), jnp.float32)

        def zero_zbuf(i, _):
            zbuf[i // (DH // 16), pl.ds((i % (DH // 16)) * 16, 16)] = zv
            return 0
        lax.fori_loop(0, ZCH * (DH // 16), zero_zbuf, 0)

        if with_cnt:
            def fill_ones(i, _):
                onesv[pl.ds(i * 16, 16)] = jnp.ones((16,), jnp.float32)
                return 0
            lax.fori_loop(0, K // 16, fill_ones, 0)

            @pl.when(s == 0)
            def _():
                def zero_cbuf(i, _):
                    cbuf[pl.ds(i * 16, 16)] = zv
                    return 0
                lax.fori_loop(0, N_NODES // 16, zero_cbuf, 0)
                pltpu.sync_copy(cbuf, cacc)

        for p, (feat, outp) in enumerate(((feat_lo, out_lo),
                                          (feat_hi, out_hi))):
            def zero_acc(j, _):
                ch = s + j * NS

                @pl.when(ch < NZC)
                def _():
                    pltpu.sync_copy(zbuf, acc.at[pl.ds(ch * ZCH, ZCH)])
                return 0
            lax.fori_loop(0, (NZC + NS - 1) // NS, zero_acc, 0)

            plsc.subcore_barrier()

            first = with_cnt and p == 0

            for pr in range(4):
                pltpu.async_copy(feat.at[srcv.at[pr]], rows.at[pr],
                                 gsem.at[pr])

            def chunk(i, _):
                b = i % 8
                rb = rows.at[b]
                pltpu.make_async_copy(feat.at[srcv.at[i]], rb,
                                      gsem.at[b]).wait()
                pltpu.async_copy(rb, acc.at[dstv.at[i]], ssem.at[b],
                                 add=True)
                if first:
                    pltpu.sync_copy(onesv, cacc.at[dstv.at[i]], add=True)

                @pl.when(i < NCH - 4)
                def _():
                    b4 = (i + 4) % 8
                    rb4 = rows.at[b4]

                    @pl.when(i >= 4)
                    def _():
                        pltpu.make_async_copy(
                            rb4, acc.at[dstv.at[i]], ssem.at[b4]).wait()
                    pltpu.async_copy(feat.at[srcv.at[i + 4]], rb4,
                                     gsem.at[b4])
                return 0
            lax.fori_loop(0, NCH, chunk, 0)

            for t in range(NCH - 8, NCH):
                pltpu.make_async_copy(rows.at[t % 8], acc.at[dstv.at[0]],
                                      ssem.at[t % 8]).wait()

            plsc.subcore_barrier()

            def export(j, _):
                ch = s + j * NS

                @pl.when(ch < NZC)
                def _():
                    r0 = ch * ZCH
                    pltpu.sync_copy(acc.at[pl.ds(r0, ZCH)], zbuf)
                    pltpu.sync_copy(zbuf, outp.at[c, pl.ds(r0, ZCH)])
                return 0
            lax.fori_loop(0, (NZC + NS - 1) // NS, export, 0)

            if p == 0:
                lax.fori_loop(0, ZCH * (DH // 16), zero_zbuf, 0)

        if with_cnt:
            @pl.when(s == 0)
            def _():
                pltpu.sync_copy(cacc, cbuf)
                pltpu.sync_copy(cbuf, cntp.at[c])

    return body


_sc_segsum_cnt = _make_sc_segsum(True)
_sc_segsum = _make_sc_segsum(False)

BR = 1000

_row_spec = pl.BlockSpec((BR, D), lambda i: (i, 0))
_half_spec = pl.BlockSpec((BR, DH), lambda i: (i, 0))
_cnt_spec = pl.BlockSpec((BR, 1), lambda i: (i, 0))
_w_spec = pl.BlockSpec((D, D), lambda i: (0, 0))
_b_spec = pl.BlockSpec((1, D), lambda i: (0, 0))


def _mean_from(lo0_ref, lo1_ref, hi0_ref, hi1_ref, c0_ref, c1_ref):
    cnt = jnp.maximum(c0_ref[...] + c1_ref[...], 1.0)
    agg = jnp.concatenate([lo0_ref[...] + lo1_ref[...],
                           hi0_ref[...] + hi1_ref[...]], axis=1)
    return agg / cnt


def _mid_body(lo0_ref, lo1_ref, hi0_ref, hi1_ref, x_ref, c0_ref, c1_ref,
              wl_ref, wr1_ref, b1_ref, wr2_ref, b2_ref,
              hlo_ref, hhi_ref, t2_ref):
    mean = _mean_from(lo0_ref, lo1_ref, hi0_ref, hi1_ref, c0_ref, c1_ref)
    t1 = jnp.dot(x_ref[...], wr1_ref[...],
                 preferred_element_type=jnp.float32) + b1_ref[...]
    h = jnp.maximum(jnp.dot(mean, wl_ref[...],
                            preferred_element_type=jnp.float32) + t1, 0.0)
    hlo_ref[...] = h[:, :DH]
    hhi_ref[...] = h[:, DH:]
    t2_ref[...] = jnp.dot(h, wr2_ref[...],
                          preferred_element_type=jnp.float32) + b2_ref[...]


def _mid(lo0, lo1, hi0, hi1, x, c0, c1, W1_l, W1_r, b1, W2_r, b2):
    return pl.pallas_call(
        _mid_body,
        grid=(N_NODES // BR,),
        in_specs=[_half_spec, _half_spec, _half_spec, _half_spec, _row_spec,
                  _cnt_spec, _cnt_spec, _w_spec, _w_spec, _b_spec,
                  _w_spec, _b_spec],
        out_specs=[_half_spec, _half_spec, _row_spec],
        out_shape=[jax.ShapeDtypeStruct((N_NODES, DH), jnp.float32),
                   jax.ShapeDtypeStruct((N_NODES, DH), jnp.float32),
                   jax.ShapeDtypeStruct((N_NODES, D), jnp.float32)],
    )(lo0, lo1, hi0, hi1, x, c0, c1, W1_l, W1_r, b1.reshape(1, D),
      W2_r, b2.reshape(1, D))


def _final_body(lo0_ref, lo1_ref, hi0_ref, hi1_ref, t_ref, c0_ref, c1_ref,
                wl_ref, o_ref):
    mean = _mean_from(lo0_ref, lo1_ref, hi0_ref, hi1_ref, c0_ref, c1_ref)
    o_ref[...] = jnp.dot(mean, wl_ref[...],
                         preferred_element_type=jnp.float32) + t_ref[...]


def _final(lo0, lo1, hi0, hi1, t, c0, c1, W_l):
    return pl.pallas_call(
        _final_body,
        grid=(N_NODES // BR,),
        in_specs=[_half_spec, _half_spec, _half_spec, _half_spec, _row_spec,
                  _cnt_spec, _cnt_spec, _w_spec],
        out_specs=_row_spec,
        out_shape=jax.ShapeDtypeStruct((N_NODES, D), jnp.float32),
    )(lo0, lo1, hi0, hi1, t, c0, c1, W_l)


def kernel(x, edge_index, W1_l, b1, W1_r, W2_l, b2, W2_r):
    src = edge_index[0].astype(jnp.int32).reshape(NW, NCH, K)
    dst = edge_index[1].astype(jnp.int32).reshape(NW, NCH, K)
    x_lo = x[:, :DH]
    x_hi = x[:, DH:]

    a_lo, a_hi, cnt = _sc_segsum_cnt(x_lo, x_hi, src, dst)
    c0 = cnt[0].reshape(N_NODES, 1)
    c1 = cnt[1].reshape(N_NODES, 1)
    h_lo, h_hi, t2 = _mid(a_lo[0], a_lo[1], a_hi[0], a_hi[1], x, c0, c1,
                          W1_l, W1_r, b1, W2_r, b2)
    b_lo, b_hi = _sc_segsum(h_lo, h_hi, src, dst)
    out = _final(b_lo[0], b_lo[1], b_hi[0], b_hi[1], t2, c0, c1, W2_l)
    return out


# speedup vs baseline: 1.2334x; 1.0580x over previous
import functools

import jax
import jax.numpy as jnp
from jax import lax
from jax.experimental import pallas as pl
from jax.experimental.pallas import tpu as pltpu
from jax.experimental.pallas import tpu_sc as plsc

N_NODES = 10000
N_EDGES = 320000
D = 128
DH = D // 2

NC = 2
NS = 16
NW = NC * NS
EPW = N_EDGES // NW
K = 80
NCH = EPW // K
ZCH = 80
NZC = N_NODES // ZCH
CCH = 2000


def _make_sc_segsum(with_cnt: bool):
    mesh = plsc.VectorSubcoreMesh(core_axis_name="c", subcore_axis_name="s")

    out_type = [jax.ShapeDtypeStruct((NC, N_NODES, DH), jnp.float32),
                jax.ShapeDtypeStruct((NC, N_NODES, DH), jnp.float32)]
    scratch = [
        pltpu.VMEM((NCH, K), jnp.int32),
        pltpu.VMEM((NCH, K), jnp.int32),
        pltpu.VMEM((12, K, DH), jnp.float32),
        pltpu.VMEM((ZCH, DH), jnp.float32),
        pltpu.VMEM_SHARED((N_NODES, DH), jnp.float32),
        pltpu.SemaphoreType.DMA((12,)),
        pltpu.SemaphoreType.DMA((12,)),
    ]
    if with_cnt:
        out_type.append(jax.ShapeDtypeStruct((NC, N_NODES), jnp.float32))
        scratch += [
            pltpu.VMEM((K,), jnp.float32),
            pltpu.VMEM((CCH,), jnp.float32),
            pltpu.VMEM_SHARED((N_NODES,), jnp.float32),
        ]

    @functools.partial(
        pl.kernel, mesh=mesh, out_type=out_type, scratch_types=scratch,
        compiler_params=pltpu.CompilerParams(use_tc_tiling_on_sc=False))
    def body(feat_lo, feat_hi, src3, dst3, *rest):
        if with_cnt:
            (out_lo, out_hi, cntp, srcv, dstv, rows, zbuf, acc, gsem, ssem,
             onesv, cbuf, cacc) = rest
        else:
            (out_lo, out_hi, srcv, dstv, rows, zbuf, acc, gsem, ssem) = rest
        c = lax.axis_index("c")
        s = lax.axis_index("s")
        wid = s * NC + c

        pltpu.sync_copy(src3.at[wid], srcv)
        pltpu.sync_copy(dst3.at[wid], dstv)

        zv = jnp.zeros((16,), jnp.float32)

        def zero_zbuf(i, _):
            zbuf[i // (DH // 16), pl.ds((i % (DH // 16)) * 16, 16)] = zv
            return 0
        lax.fori_loop(0, ZCH * (DH // 16), zero_zbuf, 0)

        if with_cnt:
            def fill_ones(i, _):
                onesv[pl.ds(i * 16, 16)] = jnp.ones((16,), jnp.float32)
                return 0
            lax.fori_loop(0, K // 16, fill_ones, 0)

            @pl.when(s == 0)
            def _():
                def zero_cbuf(i, _):
                    cbuf[pl.ds(i * 16, 16)] = zv
                    return 0
                lax.fori_loop(0, CCH // 16, zero_cbuf, 0)
                for j in range(N_NODES // CCH):
                    pltpu.sync_copy(cbuf, cacc.at[pl.ds(j * CCH, CCH)])

        for p, (feat, outp) in enumerate(((feat_lo, out_lo),
                                          (feat_hi, out_hi))):
            def zero_acc(j, _):
                ch = s + j * NS

                @pl.when(ch < NZC)
                def _():
                    pltpu.sync_copy(zbuf, acc.at[pl.ds(ch * ZCH, ZCH)])
                return 0
            lax.fori_loop(0, (NZC + NS - 1) // NS, zero_acc, 0)

            plsc.subcore_barrier()

            first = with_cnt and p == 0

            for pr in range(6):
                pltpu.async_copy(feat.at[srcv.at[pr]], rows.at[pr],
                                 gsem.at[pr])

            def chunk(i, _):
                b = i % 12
                rb = rows.at[b]
                pltpu.make_async_copy(feat.at[srcv.at[i]], rb,
                                      gsem.at[b]).wait()
                pltpu.async_copy(rb, acc.at[dstv.at[i]], ssem.at[b],
                                 add=True)
                if first:
                    pltpu.sync_copy(onesv, cacc.at[dstv.at[i]], add=True)

                @pl.when(i < NCH - 6)
                def _():
                    b4 = (i + 6) % 12
                    rb4 = rows.at[b4]

                    @pl.when(i >= 6)
                    def _():
                        pltpu.make_async_copy(
                            rb4, acc.at[dstv.at[i]], ssem.at[b4]).wait()
                    pltpu.async_copy(feat.at[srcv.at[i + 6]], rb4,
                                     gsem.at[b4])
                return 0
            lax.fori_loop(0, NCH, chunk, 0)

            for t in range(NCH - 12, NCH):
                pltpu.make_async_copy(rows.at[t % 12], acc.at[dstv.at[0]],
                                      ssem.at[t % 12]).wait()

            plsc.subcore_barrier()

            def export(j, _):
                ch = s + j * NS

                @pl.when(ch < NZC)
                def _():
                    r0 = ch * ZCH
                    pltpu.sync_copy(acc.at[pl.ds(r0, ZCH)], zbuf)
                    pltpu.sync_copy(zbuf, outp.at[c, pl.ds(r0, ZCH)])
                return 0
            lax.fori_loop(0, (NZC + NS - 1) // NS, export, 0)

            if p == 0:
                lax.fori_loop(0, ZCH * (DH // 16), zero_zbuf, 0)

        if with_cnt:
            @pl.when(s == 0)
            def _():
                for j in range(N_NODES // CCH):
                    sl = pl.ds(j * CCH, CCH)
                    pltpu.sync_copy(cacc.at[sl], cbuf)
                    pltpu.sync_copy(cbuf, cntp.at[c, sl])

    return body


_sc_segsum_cnt = _make_sc_segsum(True)
_sc_segsum = _make_sc_segsum(False)

BR = 1000

_row_spec = pl.BlockSpec((BR, D), lambda i: (i, 0))
_half_spec = pl.BlockSpec((BR, DH), lambda i: (i, 0))
_cnt_spec = pl.BlockSpec((BR, 1), lambda i: (i, 0))
_w_spec = pl.BlockSpec((D, D), lambda i: (0, 0))
_b_spec = pl.BlockSpec((1, D), lambda i: (0, 0))


def _mean_from(lo0_ref, lo1_ref, hi0_ref, hi1_ref, c0_ref, c1_ref):
    cnt = jnp.maximum(c0_ref[...] + c1_ref[...], 1.0)
    agg = jnp.concatenate([lo0_ref[...] + lo1_ref[...],
                           hi0_ref[...] + hi1_ref[...]], axis=1)
    return agg / cnt


def _mid_body(lo0_ref, lo1_ref, hi0_ref, hi1_ref, x_ref, c0_ref, c1_ref,
              wl_ref, wr1_ref, b1_ref, wr2_ref, b2_ref,
              hlo_ref, hhi_ref, t2_ref):
    mean = _mean_from(lo0_ref, lo1_ref, hi0_ref, hi1_ref, c0_ref, c1_ref)
    t1 = jnp.dot(x_ref[...], wr1_ref[...],
                 preferred_element_type=jnp.float32) + b1_ref[...]
    h = jnp.maximum(jnp.dot(mean, wl_ref[...],
                            preferred_element_type=jnp.float32) + t1, 0.0)
    hlo_ref[...] = h[:, :DH]
    hhi_ref[...] = h[:, DH:]
    t2_ref[...] = jnp.dot(h, wr2_ref[...],
                          preferred_element_type=jnp.float32) + b2_ref[...]


def _mid(lo0, lo1, hi0, hi1, x, c0, c1, W1_l, W1_r, b1, W2_r, b2):
    return pl.pallas_call(
        _mid_body,
        grid=(N_NODES // BR,),
        in_specs=[_half_spec, _half_spec, _half_spec, _half_spec, _row_spec,
                  _cnt_spec, _cnt_spec, _w_spec, _w_spec, _b_spec,
                  _w_spec, _b_spec],
        out_specs=[_half_spec, _half_spec, _row_spec],
        out_shape=[jax.ShapeDtypeStruct((N_NODES, DH), jnp.float32),
                   jax.ShapeDtypeStruct((N_NODES, DH), jnp.float32),
                   jax.ShapeDtypeStruct((N_NODES, D), jnp.float32)],
    )(lo0, lo1, hi0, hi1, x, c0, c1, W1_l, W1_r, b1.reshape(1, D),
      W2_r, b2.reshape(1, D))


def _final_body(lo0_ref, lo1_ref, hi0_ref, hi1_ref, t_ref, c0_ref, c1_ref,
                wl_ref, o_ref):
    mean = _mean_from(lo0_ref, lo1_ref, hi0_ref, hi1_ref, c0_ref, c1_ref)
    o_ref[...] = jnp.dot(mean, wl_ref[...],
                         preferred_element_type=jnp.float32) + t_ref[...]


def _final(lo0, lo1, hi0, hi1, t, c0, c1, W_l):
    return pl.pallas_call(
        _final_body,
        grid=(N_NODES // BR,),
        in_specs=[_half_spec, _half_spec, _half_spec, _half_spec, _row_spec,
                  _cnt_spec, _cnt_spec, _w_spec],
        out_specs=_row_spec,
        out_shape=jax.ShapeDtypeStruct((N_NODES, D), jnp.float32),
    )(lo0, lo1, hi0, hi1, t, c0, c1, W_l)


def kernel(x, edge_index, W1_l, b1, W1_r, W2_l, b2, W2_r):
    src = edge_index[0].astype(jnp.int32).reshape(NW, NCH, K)
    dst = edge_index[1].astype(jnp.int32).reshape(NW, NCH, K)
    x_lo = x[:, :DH]
    x_hi = x[:, DH:]

    a_lo, a_hi, cnt = _sc_segsum_cnt(x_lo, x_hi, src, dst)
    c0 = cnt[0].reshape(N_NODES, 1)
    c1 = cnt[1].reshape(N_NODES, 1)
    h_lo, h_hi, t2 = _mid(a_lo[0], a_lo[1], a_hi[0], a_hi[1], x, c0, c1,
                          W1_l, W1_r, b1, W2_r, b2)
    b_lo, b_hi = _sc_segsum(h_lo, h_hi, src, dst)
    out = _final(b_lo[0], b_lo[1], b_hi[0], b_hi[1], t2, c0, c1, W2_l)
    return out


# speedup vs baseline: 1.2431x; 1.0079x over previous
import functools

import jax
import jax.numpy as jnp
from jax import lax
from jax.experimental import pallas as pl
from jax.experimental.pallas import tpu as pltpu
from jax.experimental.pallas import tpu_sc as plsc

N_NODES = 10000
N_EDGES = 320000
D = 128
DH = D // 2

NC = 2
NS = 16
NW = NC * NS
EPW = N_EDGES // NW
K = 80
NCH = EPW // K
ZCH = 80
NZC = N_NODES // ZCH
CCH = 2000


def _make_sc_segsum(with_cnt: bool):
    mesh = plsc.VectorSubcoreMesh(core_axis_name="c", subcore_axis_name="s")

    out_type = [jax.ShapeDtypeStruct((NC, N_NODES, DH), jnp.float32),
                jax.ShapeDtypeStruct((NC, N_NODES, DH), jnp.float32)]
    scratch = [
        pltpu.VMEM((NCH, K), jnp.int32),
        pltpu.VMEM((NCH, K), jnp.int32),
        pltpu.VMEM((12, K, DH), jnp.float32),
        pltpu.VMEM((ZCH, DH), jnp.float32),
        pltpu.VMEM_SHARED((N_NODES, DH), jnp.float32),
        pltpu.SemaphoreType.DMA((12,)),
        pltpu.SemaphoreType.DMA((12,)),
    ]
    if with_cnt:
        out_type.append(jax.ShapeDtypeStruct((NC, N_NODES), jnp.float32))
        scratch += [
            pltpu.VMEM((K,), jnp.float32),
            pltpu.VMEM((CCH,), jnp.float32),
            pltpu.VMEM_SHARED((N_NODES,), jnp.float32),
        ]

    @functools.partial(
        pl.kernel, mesh=mesh, out_type=out_type, scratch_types=scratch,
        compiler_params=pltpu.CompilerParams(use_tc_tiling_on_sc=False))
    def body(feat_lo, feat_hi, src3, dst3, *rest):
        if with_cnt:
            (out_lo, out_hi, cntp, srcv, dstv, rows, zbuf, acc, gsem, ssem,
             onesv, cbuf, cacc) = rest
        else:
            (out_lo, out_hi, srcv, dstv, rows, zbuf, acc, gsem, ssem) = rest
        c = lax.axis_index("c")
        s = lax.axis_index("s")
        wid = s * NC + c

        pltpu.sync_copy(src3.at[wid], srcv)
        pltpu.sync_copy(dst3.at[wid], dstv)

        zv = jnp.zeros((16,), jnp.float32)

        def zero_zbuf(i, _):
            zbuf[i // (DH // 16), pl.ds((i % (DH // 16)) * 16, 16)] = zv
            return 0
        lax.fori_loop(0, ZCH * (DH // 16), zero_zbuf, 0)

        if with_cnt:
            def fill_ones(i, _):
                onesv[pl.ds(i * 16, 16)] = jnp.ones((16,), jnp.float32)
                return 0
            lax.fori_loop(0, K // 16, fill_ones, 0)

            @pl.when(s == 0)
            def _():
                def zero_cbuf(i, _):
                    cbuf[pl.ds(i * 16, 16)] = zv
                    return 0
                lax.fori_loop(0, CCH // 16, zero_cbuf, 0)
                for j in range(N_NODES // CCH):
                    pltpu.sync_copy(cbuf, cacc.at[pl.ds(j * CCH, CCH)])

        for p, (feat, outp) in enumerate(((feat_lo, out_lo),
                                          (feat_hi, out_hi))):
            def zero_acc(j, _):
                ch = s + j * NS

                @pl.when(ch < NZC)
                def _():
                    pltpu.sync_copy(zbuf, acc.at[pl.ds(ch * ZCH, ZCH)])
                return 0
            lax.fori_loop(0, (NZC + NS - 1) // NS, zero_acc, 0)

            plsc.subcore_barrier()

            first = with_cnt and p == 0

            if p == 0:
                for pr in range(6):
                    pltpu.async_copy(feat.at[srcv.at[pr]], rows.at[pr],
                                     gsem.at[pr])

            def chunk(i, _):
                b = i % 12
                rb = rows.at[b]
                pltpu.make_async_copy(feat.at[srcv.at[i]], rb,
                                      gsem.at[b]).wait()
                pltpu.async_copy(rb, acc.at[dstv.at[i]], ssem.at[b],
                                 add=True)
                if first:
                    pltpu.sync_copy(onesv, cacc.at[dstv.at[i]], add=True)

                @pl.when(i < NCH - 6)
                def _():
                    b4 = (i + 6) % 12
                    rb4 = rows.at[b4]

                    @pl.when(i >= 6)
                    def _():
                        pltpu.make_async_copy(
                            rb4, acc.at[dstv.at[i]], ssem.at[b4]).wait()
                    pltpu.async_copy(feat.at[srcv.at[i + 6]], rb4,
                                     gsem.at[b4])
                return 0
            lax.fori_loop(0, NCH, chunk, 0)

            for t in range(NCH - 12, NCH):
                pltpu.make_async_copy(rows.at[t % 12], acc.at[dstv.at[0]],
                                      ssem.at[t % 12]).wait()

            if p == 0:
                for pr in range(6):
                    pltpu.async_copy(feat_hi.at[srcv.at[pr]], rows.at[pr],
                                     gsem.at[pr])

            plsc.subcore_barrier()

            def export(j, _):
                ch = s + j * NS

                @pl.when(ch < NZC)
                def _():
                    r0 = ch * ZCH
                    pltpu.sync_copy(acc.at[pl.ds(r0, ZCH)], zbuf)
                    pltpu.sync_copy(zbuf, outp.at[c, pl.ds(r0, ZCH)])
                return 0
            lax.fori_loop(0, (NZC + NS - 1) // NS, export, 0)

            if p == 0:
                lax.fori_loop(0, ZCH * (DH // 16), zero_zbuf, 0)

        if with_cnt:
            @pl.when(s == 0)
            def _():
                for j in range(N_NODES // CCH):
                    sl = pl.ds(j * CCH, CCH)
                    pltpu.sync_copy(cacc.at[sl], cbuf)
                    pltpu.sync_copy(cbuf, cntp.at[c, sl])

    return body


_sc_segsum_cnt = _make_sc_segsum(True)
_sc_segsum = _make_sc_segsum(False)

BR = 1000

_row_spec = pl.BlockSpec((BR, D), lambda i: (i, 0))
_half_spec = pl.BlockSpec((BR, DH), lambda i: (i, 0))
_cnt_spec = pl.BlockSpec((BR, 1), lambda i: (i, 0))
_w_spec = pl.BlockSpec((D, D), lambda i: (0, 0))
_b_spec = pl.BlockSpec((1, D), lambda i: (0, 0))


def _mean_from(lo0_ref, lo1_ref, hi0_ref, hi1_ref, c0_ref, c1_ref):
    cnt = jnp.maximum(c0_ref[...] + c1_ref[...], 1.0)
    agg = jnp.concatenate([lo0_ref[...] + lo1_ref[...],
                           hi0_ref[...] + hi1_ref[...]], axis=1)
    return agg / cnt


def _mid_body(lo0_ref, lo1_ref, hi0_ref, hi1_ref, x_ref, c0_ref, c1_ref,
              wl_ref, wr1_ref, b1_ref, wr2_ref, b2_ref,
              hlo_ref, hhi_ref, t2_ref):
    mean = _mean_from(lo0_ref, lo1_ref, hi0_ref, hi1_ref, c0_ref, c1_ref)
    t1 = jnp.dot(x_ref[...], wr1_ref[...],
                 preferred_element_type=jnp.float32) + b1_ref[...]
    h = jnp.maximum(jnp.dot(mean, wl_ref[...],
                            preferred_element_type=jnp.float32) + t1, 0.0)
    hlo_ref[...] = h[:, :DH]
    hhi_ref[...] = h[:, DH:]
    t2_ref[...] = jnp.dot(h, wr2_ref[...],
                          preferred_element_type=jnp.float32) + b2_ref[...]


def _mid(lo0, lo1, hi0, hi1, x, c0, c1, W1_l, W1_r, b1, W2_r, b2):
    return pl.pallas_call(
        _mid_body,
        grid=(N_NODES // BR,),
        in_specs=[_half_spec, _half_spec, _half_spec, _half_spec, _row_spec,
                  _cnt_spec, _cnt_spec, _w_spec, _w_spec, _b_spec,
                  _w_spec, _b_spec],
        out_specs=[_half_spec, _half_spec, _row_spec],
        out_shape=[jax.ShapeDtypeStruct((N_NODES, DH), jnp.float32),
                   jax.ShapeDtypeStruct((N_NODES, DH), jnp.float32),
                   jax.ShapeDtypeStruct((N_NODES, D), jnp.float32)],
    )(lo0, lo1, hi0, hi1, x, c0, c1, W1_l, W1_r, b1.reshape(1, D),
      W2_r, b2.reshape(1, D))


def _final_body(lo0_ref, lo1_ref, hi0_ref, hi1_ref, t_ref, c0_ref, c1_ref,
                wl_ref, o_ref):
    mean = _mean_from(lo0_ref, lo1_ref, hi0_ref, hi1_ref, c0_ref, c1_ref)
    o_ref[...] = jnp.dot(mean, wl_ref[...],
                         preferred_element_type=jnp.float32) + t_ref[...]


def _final(lo0, lo1, hi0, hi1, t, c0, c1, W_l):
    return pl.pallas_call(
        _final_body,
        grid=(N_NODES // BR,),
        in_specs=[_half_spec, _half_spec, _half_spec, _half_spec, _row_spec,
                  _cnt_spec, _cnt_spec, _w_spec],
        out_specs=_row_spec,
        out_shape=jax.ShapeDtypeStruct((N_NODES, D), jnp.float32),
    )(lo0, lo1, hi0, hi1, t, c0, c1, W_l)


def kernel(x, edge_index, W1_l, b1, W1_r, W2_l, b2, W2_r):
    src = edge_index[0].astype(jnp.int32).reshape(NW, NCH, K)
    dst = edge_index[1].astype(jnp.int32).reshape(NW, NCH, K)
    x_lo = x[:, :DH]
    x_hi = x[:, DH:]

    a_lo, a_hi, cnt = _sc_segsum_cnt(x_lo, x_hi, src, dst)
    c0 = cnt[0].reshape(N_NODES, 1)
    c1 = cnt[1].reshape(N_NODES, 1)
    h_lo, h_hi, t2 = _mid(a_lo[0], a_lo[1], a_hi[0], a_hi[1], x, c0, c1,
                          W1_l, W1_r, b1, W2_r, b2)
    b_lo, b_hi = _sc_segsum(h_lo, h_hi, src, dst)
    out = _final(b_lo[0], b_lo[1], b_hi[0], b_hi[1], t2, c0, c1, W2_l)
    return out
